# Initial kernel scaffold; baseline (speedup 1.0000x reference)
import jax
import jax.numpy as jnp
from jax import lax
from jax.experimental import pallas as pl
from jax.experimental.pallas import tpu as pltpu
from jax.experimental.pallas import tpu_sc as plsc

B = 16
MAX_LEN = 2048
TOTAL = 16384
HALF = MAX_LEN // 2
NEG_INF = jnp.float32(-jnp.inf)


def _body(flat_hbm, cu_hbm, out_hbm, flat_v, cu_v, buf_v):
    c = lax.axis_index("c")
    s = lax.axis_index("s")

    pltpu.sync_copy(cu_hbm, cu_v)
    pltpu.sync_copy(flat_hbm, flat_v)

    row_vec = jnp.full((16,), s, dtype=jnp.int32)
    cu_i = plsc.load_gather(cu_v, [row_vec])
    cu_i1 = plsc.load_gather(cu_v, [row_vec + 1])

    c0 = c * HALF
    lanes = lax.iota(jnp.int32, 16)

    def step(t, carry):
        cols = c0 + t * 16 + lanes
        idx = cu_i + cols
        valid = idx < cu_i1
        v = plsc.load_gather(flat_v, [jnp.minimum(idx, TOTAL - 1)])
        buf_v[pl.ds(t * 16, 16)] = jnp.where(valid & (v != 0.0), v, NEG_INF)
        return carry

    lax.fori_loop(0, HALF // 16, step, 0)

    pltpu.sync_copy(buf_v, out_hbm.at[s, pl.ds(c0, HALF)])


def kernel(flat, cu_seqlens):
    cu_pad = jnp.zeros((32,), jnp.int32).at[: B + 1].set(cu_seqlens)
    mesh = plsc.VectorSubcoreMesh(
        core_axis_name="c", subcore_axis_name="s", num_cores=2, num_subcores=16
    )
    run = pl.kernel(
        _body,
        out_type=jax.ShapeDtypeStruct((B, MAX_LEN), jnp.float32),
        mesh=mesh,
        scratch_types=[
            pltpu.VMEM((TOTAL,), jnp.float32),
            pltpu.VMEM((32,), jnp.int32),
            pltpu.VMEM((HALF,), jnp.float32),
        ],
    )
    return run(flat, cu_pad)


# speedup vs baseline: 7.7448x; 7.7448x over previous
import jax
import jax.numpy as jnp
import numpy as np
from jax import lax
from jax.experimental import pallas as pl
from jax.experimental.pallas import tpu as pltpu
from jax.experimental.pallas import tpu_sc as plsc

B = 16
MAX_LEN = 2048
TOTAL = 16384
HALF = MAX_LEN // 2
NEG_INF = np.float32(-np.inf)


def _body(flat_hbm, cu_hbm, out_hbm, flat_v, cu_v, buf_v):
    c = lax.axis_index("c")
    s = lax.axis_index("s")

    pltpu.sync_copy(cu_hbm, cu_v)
    pltpu.sync_copy(flat_hbm, flat_v)

    row_vec = jnp.full((16,), s, dtype=jnp.int32)
    cu_i = plsc.load_gather(cu_v, [row_vec])
    cu_i1 = plsc.load_gather(cu_v, [row_vec + 1])

    c0 = c * HALF
    lanes = lax.iota(jnp.int32, 16)

    def step(t, carry):
        cols = c0 + t * 16 + lanes
        idx = cu_i + cols
        valid = idx < cu_i1
        v = plsc.load_gather(flat_v, [jnp.minimum(idx, TOTAL - 1)])
        buf_v[pl.ds(t * 16, 16)] = jnp.where(valid & (v != 0.0), v, NEG_INF)
        return carry

    lax.fori_loop(0, HALF // 16, step, 0)

    pltpu.sync_copy(buf_v, out_hbm.at[s, pl.ds(c0, HALF)])


def kernel(flat, cu_seqlens):
    cu_pad = jnp.zeros((32,), jnp.int32).at[: B + 1].set(cu_seqlens)
    mesh = plsc.VectorSubcoreMesh(
        core_axis_name="c", subcore_axis_name="s", num_cores=2, num_subcores=16
    )
    run = pl.kernel(
        _body,
        out_type=jax.ShapeDtypeStruct((B, MAX_LEN), jnp.float32),
        mesh=mesh,
        scratch_types=[
            pltpu.VMEM((TOTAL,), jnp.float32),
            pltpu.VMEM((32,), jnp.int32),
            pltpu.VMEM((HALF,), jnp.float32),
        ],
        compiler_params=pltpu.CompilerParams(needs_layout_passes=False),
    )
    return run(flat, cu_pad)
